# baseline (speedup 1.0000x reference)
import jax
import jax.numpy as jnp
from jax import lax
from jax.experimental import pallas as pl
from jax.experimental.pallas import tpu as pltpu
from jax.experimental.pallas import tpu_sc as plsc

N = 8192
NB = 16
LANES = 16
NUM_WORKERS = 32
PTS_PER_W = N // NUM_WORKERS
SUB = 4
BIG = float(1e30)


def _bf16_round(v):
    u = plsc.bitcast(v, jnp.uint32)
    lsb = (u >> jnp.uint32(16)) & jnp.uint32(1)
    r = (u + jnp.uint32(0x7FFF) + lsb) & jnp.uint32(0xFFFF0000)
    return plsc.bitcast(r, jnp.float32)


def _lower_bound(ref, targets, n):
    lo = jnp.zeros((LANES,), jnp.int32)
    hi = jnp.full((LANES,), n, jnp.int32)

    def body(_, carry):
        lo, hi = carry
        active = lo < hi
        mid = (lo + hi) >> 1
        midc = jnp.minimum(mid, n - 1)
        v = plsc.load_gather(ref, [midc])
        go_right = active & (v < targets)
        lo = jnp.where(go_right, mid + 1, lo)
        hi = jnp.where(active & jnp.logical_not(go_right), mid, hi)
        return lo, hi

    lo, hi = lax.fori_loop(0, 14, body, (lo, hi))
    return lo


def _splat(vec, l):
    return jnp.full((LANES,), vec[l], vec.dtype)


def _scan_points(base, bpt_ref, pox, poy, poz,
                 aox, aoy, aoz, ab_ref,
                 seg_s_ref, seg_e_ref, out_ref,
                 drx, dry, drz, dnrm):
    lane = lax.iota(jnp.int32, LANES)

    bv_first = bpt_ref[pl.ds(base, LANES)]
    bv_last = bpt_ref[pl.ds(base + PTS_PER_W - LANES, LANES)]
    b_first = _splat(bv_first, 0)
    b_last = _splat(bv_last, LANES - 1)
    s_tile = jnp.min(plsc.load_gather(seg_s_ref, [b_first]))
    e_tile = jnp.max(plsc.load_gather(seg_e_ref, [b_last]))

    def pre_body(c, _):
        off = c * LANES
        a = aox[pl.ds(off, LANES)]
        b = aoy[pl.ds(off, LANES)]
        d = aoz[pl.ds(off, LANES)]
        drx[pl.ds(off, LANES)] = _bf16_round(a)
        dry[pl.ds(off, LANES)] = _bf16_round(b)
        drz[pl.ds(off, LANES)] = _bf16_round(d)
        dnrm[pl.ds(off, LANES)] = a * a + b * b + d * d
        return 0

    lax.fori_loop(s_tile >> 4, (e_tile + LANES - 1) >> 4, pre_body, 0)

    def group_body(g, _):
        goff = base + g * LANES
        bv = bpt_ref[pl.ds(goff, LANES)]
        pxo = pox[pl.ds(goff, LANES)]
        pyo = poy[pl.ds(goff, LANES)]
        pzo = poz[pl.ds(goff, LANES)]
        x2v = pxo * pxo + pyo * pyo + pzo * pzo
        p2x = _bf16_round(pxo)
        p2x = p2x + p2x
        p2y = _bf16_round(pyo)
        p2y = p2y + p2y
        p2z = _bf16_round(pzo)
        p2z = p2z + p2z
        sv = plsc.load_gather(seg_s_ref, [bv])
        ev = plsc.load_gather(seg_e_ref, [bv])

        resv = jnp.zeros((LANES,), jnp.float32)
        for sb in range(LANES // SUB):
            lo_l = sb * SUB
            hi_l = sb * SUB + SUB - 1
            splats = []
            for j in range(SUB):
                l = lo_l + j
                splats.append((_splat(p2x, l), _splat(p2y, l),
                               _splat(p2z, l), _splat(bv, l)))
            s_u = sv[lo_l]
            e_u = ev[hi_l]
            s_i = sv[hi_l]
            e_i = ev[lo_l]
            c0 = s_u >> 4
            c1 = (e_u + LANES - 1) >> 4
            i0 = jnp.clip((s_i + LANES - 1) >> 4, c0, c1)
            i1 = jnp.clip(e_i >> 4, i0, c1)
            n2 = (i1 - i0) >> 1
            i1p = i0 + n2 * 2

            def masked_body(c, accs, splats=splats):
                off = c * LANES
                rxc = drx[pl.ds(off, LANES)]
                ryc = dry[pl.ds(off, LANES)]
                rzc = drz[pl.ds(off, LANES)]
                y2c = dnrm[pl.ds(off, LANES)]
                abc = ab_ref[pl.ds(off, LANES)]
                out = []
                for j in range(SUB):
                    sx, sy, sz, sb_ = splats[j]
                    q = ((y2c - sx * rxc) - sy * ryc) - sz * rzc
                    m = abc == sb_
                    out.append(jnp.minimum(
                        accs[j], jnp.where(m, q, jnp.float32(BIG))))
                return tuple(out)

            def open_chunk(off, accs, splats=splats):
                rxc = drx[pl.ds(off, LANES)]
                ryc = dry[pl.ds(off, LANES)]
                rzc = drz[pl.ds(off, LANES)]
                y2c = dnrm[pl.ds(off, LANES)]
                out = []
                for j in range(SUB):
                    sx, sy, sz, _ = splats[j]
                    q = ((y2c - sx * rxc) - sy * ryc) - sz * rzc
                    out.append(jnp.minimum(accs[j], q))
                return tuple(out)

            def open2_body(i, accs, i0=i0):
                off = (i0 + i * 2) * LANES
                accs = open_chunk(off, accs)
                return open_chunk(off + LANES, accs)

            init = tuple(jnp.full((LANES,), BIG, jnp.float32)
                         for _ in range(SUB))
            accs = lax.fori_loop(c0, i0, masked_body, init)
            accs = lax.fori_loop(0, n2, open2_body, accs)
            accs = lax.fori_loop(i1p, c1, masked_body, accs)
            for j in range(SUB):
                l = lo_l + j
                val = jnp.maximum(x2v[l] + jnp.min(accs[j]), 0.0)
                resv = jnp.where(lane == l, val, resv)

        out_ref[pl.ds(g * LANES, LANES)] = resv
        return 0

    lax.fori_loop(0, PTS_PER_W // LANES, group_body, 0)


def _chamfer_body(xx_h, xy_h, xz_h, yx_h, yy_h, yz_h, bx_h, by_h,
                  dx_out, dy_out, degx_out, degy_out,
                  xx, xy, xz, yx, yy, yz, bx, by,
                  drx, dry, drz, dnrm,
                  dxs, dys, vys, vye, vxs, vxe, vdeg, sem):
    wid = lax.axis_index("s") * 2 + lax.axis_index("c")
    base = wid * PTS_PER_W

    cb = pltpu.async_copy(bx_h, bx, sem)
    cb2 = pltpu.async_copy(by_h, by, sem)
    c1 = pltpu.async_copy(xx_h, xx, sem)
    c2 = pltpu.async_copy(xy_h, xy, sem)
    c3 = pltpu.async_copy(xz_h, xz, sem)
    c4 = pltpu.async_copy(yx_h, yx, sem)
    c5 = pltpu.async_copy(yy_h, yy, sem)
    c6 = pltpu.async_copy(yz_h, yz, sem)
    cb.wait()
    cb2.wait()

    lane = lax.iota(jnp.int32, LANES)
    ys = _lower_bound(by, lane, N)
    xs = _lower_bound(bx, lane, N)
    vys[...] = ys
    vxs[...] = xs
    lshift = jnp.minimum(lane + 1, NB - 1)
    last = lane == NB - 1
    ye = jnp.where(last, N, plsc.load_gather(vys, [lshift]))
    xe = jnp.where(last, N, plsc.load_gather(vxs, [lshift]))
    vye[...] = ye
    vxe[...] = xe
    c1.wait()
    c2.wait()
    c3.wait()
    c4.wait()
    c5.wait()
    c6.wait()

    @pl.when(wid == 0)
    def _():
        vdeg[...] = xe - xs
        pltpu.sync_copy(vdeg, degx_out)
        vdeg[...] = ye - ys
        pltpu.sync_copy(vdeg, degy_out)

    _scan_points(base, bx, xx, xy, xz, yx, yy, yz, by, vys, vye, dxs,
                 drx, dry, drz, dnrm)
    cdx = pltpu.async_copy(dxs, dx_out.at[pl.ds(base, PTS_PER_W)], sem)

    _scan_points(base, by, yx, yy, yz, xx, xy, xz, bx, vxs, vxe, dys,
                 drx, dry, drz, dnrm)
    pltpu.sync_copy(dys, dy_out.at[pl.ds(base, PTS_PER_W)])
    cdx.wait()


def _make_sc_kernel():
    mesh = plsc.VectorSubcoreMesh(core_axis_name="c", subcore_axis_name="s",
                                  num_cores=2, num_subcores=16)
    out_type = (
        jax.ShapeDtypeStruct((N,), jnp.float32),
        jax.ShapeDtypeStruct((N,), jnp.float32),
        jax.ShapeDtypeStruct((NB,), jnp.int32),
        jax.ShapeDtypeStruct((NB,), jnp.int32),
    )
    scratch = (
        [pltpu.VMEM((N,), jnp.float32) for _ in range(6)]
        + [pltpu.VMEM((N,), jnp.int32), pltpu.VMEM((N,), jnp.int32)]
        + [pltpu.VMEM((N,), jnp.float32) for _ in range(4)]
        + [pltpu.VMEM((PTS_PER_W,), jnp.float32),
           pltpu.VMEM((PTS_PER_W,), jnp.float32)]
        + [pltpu.VMEM((LANES,), jnp.int32) for _ in range(5)]
        + [pltpu.SemaphoreType.DMA]
    )
    return pl.kernel(_chamfer_body, out_type=out_type, mesh=mesh,
                     scratch_types=scratch,
                     compiler_params=pltpu.CompilerParams(
                         needs_layout_passes=False))


_sc_kernel_cache = []


def _get_sc_kernel():
    if not _sc_kernel_cache:
        _sc_kernel_cache.append(_make_sc_kernel())
    return _sc_kernel_cache[0]


def kernel(x, y, batch_x, batch_y):
    x = jnp.asarray(x, jnp.float32)
    y = jnp.asarray(y, jnp.float32)
    bx = batch_x.astype(jnp.int32)
    by = batch_y.astype(jnp.int32)
    dist_x, dist_y, deg_x, deg_y = _get_sc_kernel()(
        x[:, 0], x[:, 1], x[:, 2], y[:, 0], y[:, 1], y[:, 2], bx, by)
    return dist_x, dist_y, deg_x, deg_y


# speedup vs baseline: 1.0181x; 1.0181x over previous
import jax
import jax.numpy as jnp
from jax import lax
from jax.experimental import pallas as pl
from jax.experimental.pallas import tpu as pltpu
from jax.experimental.pallas import tpu_sc as plsc

N = 8192
NB = 16
LANES = 16
NUM_WORKERS = 32
PTS_PER_W = N // NUM_WORKERS
SUB = 8
BIG = float(1e30)


def _bf16_round(v):
    u = plsc.bitcast(v, jnp.uint32)
    lsb = (u >> jnp.uint32(16)) & jnp.uint32(1)
    r = (u + jnp.uint32(0x7FFF) + lsb) & jnp.uint32(0xFFFF0000)
    return plsc.bitcast(r, jnp.float32)


def _lower_bound(ref, targets, n):
    lo = jnp.zeros((LANES,), jnp.int32)
    hi = jnp.full((LANES,), n, jnp.int32)

    def body(_, carry):
        lo, hi = carry
        active = lo < hi
        mid = (lo + hi) >> 1
        midc = jnp.minimum(mid, n - 1)
        v = plsc.load_gather(ref, [midc])
        go_right = active & (v < targets)
        lo = jnp.where(go_right, mid + 1, lo)
        hi = jnp.where(active & jnp.logical_not(go_right), mid, hi)
        return lo, hi

    lo, hi = lax.fori_loop(0, 14, body, (lo, hi))
    return lo


def _splat(vec, l):
    return jnp.full((LANES,), vec[l], vec.dtype)


def _scan_points(base, bpt_ref, pox, poy, poz,
                 aox, aoy, aoz, ab_ref,
                 seg_s_ref, seg_e_ref, out_ref,
                 drx, dry, drz, dnrm):
    lane = lax.iota(jnp.int32, LANES)

    bv_first = bpt_ref[pl.ds(base, LANES)]
    bv_last = bpt_ref[pl.ds(base + PTS_PER_W - LANES, LANES)]
    b_first = _splat(bv_first, 0)
    b_last = _splat(bv_last, LANES - 1)
    s_tile = jnp.min(plsc.load_gather(seg_s_ref, [b_first]))
    e_tile = jnp.max(plsc.load_gather(seg_e_ref, [b_last]))

    def pre_body(c, _):
        off = c * LANES
        a = aox[pl.ds(off, LANES)]
        b = aoy[pl.ds(off, LANES)]
        d = aoz[pl.ds(off, LANES)]
        drx[pl.ds(off, LANES)] = _bf16_round(a)
        dry[pl.ds(off, LANES)] = _bf16_round(b)
        drz[pl.ds(off, LANES)] = _bf16_round(d)
        dnrm[pl.ds(off, LANES)] = a * a + b * b + d * d
        return 0

    lax.fori_loop(s_tile >> 4, (e_tile + LANES - 1) >> 4, pre_body, 0)

    def group_body(g, _):
        goff = base + g * LANES
        bv = bpt_ref[pl.ds(goff, LANES)]
        pxo = pox[pl.ds(goff, LANES)]
        pyo = poy[pl.ds(goff, LANES)]
        pzo = poz[pl.ds(goff, LANES)]
        x2v = pxo * pxo + pyo * pyo + pzo * pzo
        p2x = _bf16_round(pxo)
        p2x = p2x + p2x
        p2y = _bf16_round(pyo)
        p2y = p2y + p2y
        p2z = _bf16_round(pzo)
        p2z = p2z + p2z
        sv = plsc.load_gather(seg_s_ref, [bv])
        ev = plsc.load_gather(seg_e_ref, [bv])

        resv = jnp.zeros((LANES,), jnp.float32)
        for sb in range(LANES // SUB):
            lo_l = sb * SUB
            hi_l = sb * SUB + SUB - 1
            splats = []
            for j in range(SUB):
                l = lo_l + j
                splats.append((_splat(p2x, l), _splat(p2y, l),
                               _splat(p2z, l), _splat(bv, l)))
            s_u = sv[lo_l]
            e_u = ev[hi_l]
            s_i = sv[hi_l]
            e_i = ev[lo_l]
            c0 = s_u >> 4
            c1 = (e_u + LANES - 1) >> 4
            i0 = jnp.clip((s_i + LANES - 1) >> 4, c0, c1)
            i1 = jnp.clip(e_i >> 4, i0, c1)
            n2 = (i1 - i0) >> 1
            i1p = i0 + n2 * 2

            def masked_body(c, accs, splats=splats):
                off = c * LANES
                rxc = drx[pl.ds(off, LANES)]
                ryc = dry[pl.ds(off, LANES)]
                rzc = drz[pl.ds(off, LANES)]
                y2c = dnrm[pl.ds(off, LANES)]
                abc = ab_ref[pl.ds(off, LANES)]
                out = []
                for j in range(SUB):
                    sx, sy, sz, sb_ = splats[j]
                    q = ((y2c - sx * rxc) - sy * ryc) - sz * rzc
                    m = abc == sb_
                    out.append(jnp.minimum(
                        accs[j], jnp.where(m, q, jnp.float32(BIG))))
                return tuple(out)

            def open_chunk(off, accs, splats=splats):
                rxc = drx[pl.ds(off, LANES)]
                ryc = dry[pl.ds(off, LANES)]
                rzc = drz[pl.ds(off, LANES)]
                y2c = dnrm[pl.ds(off, LANES)]
                out = []
                for j in range(SUB):
                    sx, sy, sz, _ = splats[j]
                    q = ((y2c - sx * rxc) - sy * ryc) - sz * rzc
                    out.append(jnp.minimum(accs[j], q))
                return tuple(out)

            def open2_body(i, accs, i0=i0):
                off = (i0 + i * 2) * LANES
                accs = open_chunk(off, accs)
                return open_chunk(off + LANES, accs)

            init = tuple(jnp.full((LANES,), BIG, jnp.float32)
                         for _ in range(SUB))
            accs = lax.fori_loop(c0, i0, masked_body, init)
            accs = lax.fori_loop(0, n2, open2_body, accs)
            accs = lax.fori_loop(i1p, c1, masked_body, accs)
            for j in range(SUB):
                l = lo_l + j
                val = jnp.maximum(x2v[l] + jnp.min(accs[j]), 0.0)
                resv = jnp.where(lane == l, val, resv)

        out_ref[pl.ds(g * LANES, LANES)] = resv
        return 0

    lax.fori_loop(0, PTS_PER_W // LANES, group_body, 0)


def _chamfer_body(xx_h, xy_h, xz_h, yx_h, yy_h, yz_h, bx_h, by_h,
                  dx_out, dy_out, degx_out, degy_out,
                  xx, xy, xz, yx, yy, yz, bx, by,
                  drx, dry, drz, dnrm,
                  dxs, dys, vys, vye, vxs, vxe, vdeg, sem):
    wid = lax.axis_index("s") * 2 + lax.axis_index("c")
    base = wid * PTS_PER_W

    cb = pltpu.async_copy(bx_h, bx, sem)
    cb2 = pltpu.async_copy(by_h, by, sem)
    c1 = pltpu.async_copy(xx_h, xx, sem)
    c2 = pltpu.async_copy(xy_h, xy, sem)
    c3 = pltpu.async_copy(xz_h, xz, sem)
    c4 = pltpu.async_copy(yx_h, yx, sem)
    c5 = pltpu.async_copy(yy_h, yy, sem)
    c6 = pltpu.async_copy(yz_h, yz, sem)
    cb.wait()
    cb2.wait()

    lane = lax.iota(jnp.int32, LANES)
    ys = _lower_bound(by, lane, N)
    xs = _lower_bound(bx, lane, N)
    vys[...] = ys
    vxs[...] = xs
    lshift = jnp.minimum(lane + 1, NB - 1)
    last = lane == NB - 1
    ye = jnp.where(last, N, plsc.load_gather(vys, [lshift]))
    xe = jnp.where(last, N, plsc.load_gather(vxs, [lshift]))
    vye[...] = ye
    vxe[...] = xe
    c1.wait()
    c2.wait()
    c3.wait()
    c4.wait()
    c5.wait()
    c6.wait()

    @pl.when(wid == 0)
    def _():
        vdeg[...] = xe - xs
        pltpu.sync_copy(vdeg, degx_out)
        vdeg[...] = ye - ys
        pltpu.sync_copy(vdeg, degy_out)

    _scan_points(base, bx, xx, xy, xz, yx, yy, yz, by, vys, vye, dxs,
                 drx, dry, drz, dnrm)
    cdx = pltpu.async_copy(dxs, dx_out.at[pl.ds(base, PTS_PER_W)], sem)

    _scan_points(base, by, yx, yy, yz, xx, xy, xz, bx, vxs, vxe, dys,
                 drx, dry, drz, dnrm)
    pltpu.sync_copy(dys, dy_out.at[pl.ds(base, PTS_PER_W)])
    cdx.wait()


def _make_sc_kernel():
    mesh = plsc.VectorSubcoreMesh(core_axis_name="c", subcore_axis_name="s",
                                  num_cores=2, num_subcores=16)
    out_type = (
        jax.ShapeDtypeStruct((N,), jnp.float32),
        jax.ShapeDtypeStruct((N,), jnp.float32),
        jax.ShapeDtypeStruct((NB,), jnp.int32),
        jax.ShapeDtypeStruct((NB,), jnp.int32),
    )
    scratch = (
        [pltpu.VMEM((N,), jnp.float32) for _ in range(6)]
        + [pltpu.VMEM((N,), jnp.int32), pltpu.VMEM((N,), jnp.int32)]
        + [pltpu.VMEM((N,), jnp.float32) for _ in range(4)]
        + [pltpu.VMEM((PTS_PER_W,), jnp.float32),
           pltpu.VMEM((PTS_PER_W,), jnp.float32)]
        + [pltpu.VMEM((LANES,), jnp.int32) for _ in range(5)]
        + [pltpu.SemaphoreType.DMA]
    )
    return pl.kernel(_chamfer_body, out_type=out_type, mesh=mesh,
                     scratch_types=scratch,
                     compiler_params=pltpu.CompilerParams(
                         needs_layout_passes=False))


_sc_kernel_cache = []


def _get_sc_kernel():
    if not _sc_kernel_cache:
        _sc_kernel_cache.append(_make_sc_kernel())
    return _sc_kernel_cache[0]


def kernel(x, y, batch_x, batch_y):
    x = jnp.asarray(x, jnp.float32)
    y = jnp.asarray(y, jnp.float32)
    bx = batch_x.astype(jnp.int32)
    by = batch_y.astype(jnp.int32)
    dist_x, dist_y, deg_x, deg_y = _get_sc_kernel()(
        x[:, 0], x[:, 1], x[:, 2], y[:, 0], y[:, 1], y[:, 2], bx, by)
    return dist_x, dist_y, deg_x, deg_y
